# Initial kernel scaffold; baseline (speedup 1.0000x reference)
import jax
import jax.numpy as jnp
from jax.experimental import pallas as pl


def kernel(x, pos, edge_index, W_in, b_in, W_out, b_out, W_lin, W_src, W_dst, pos_W1, pos_b1, pos_W2, pos_b2, attn_W1, attn_b1, attn_W2, attn_b2):
    raise NotImplementedError("write your pallas kernel here")


# speedup vs baseline: 4.3202x; 4.3202x over previous
import functools

import jax
import jax.numpy as jnp
from jax import lax
from jax.experimental import pallas as pl
from jax.experimental.pallas import tpu as pltpu
from jax.experimental.pallas import tpu_sc as plsc

N = 10000
D = 128
H = 64
E = 320000
EL = E + N

NP_ = 10240
NB0 = 512
EP = 335872
IDX_ROWS = EP // 128
NBLK = 41
B1 = 256
B3 = 256
NBLK3 = 82
ACC_ROWS = NP_

_prec = jax.lax.Precision.HIGHEST


def _prologue_body(xb, posb, W_in, b_in, W_dst, W_src, attn_W1, attn_b1,
                   pos_W1p, pos_b1, W_lin, vd_o, vsn_o, msg_o):
    h = jax.nn.relu(jnp.dot(xb[...], W_in[...], precision=_prec) + b_in[...])
    u = jnp.dot(jnp.dot(h, W_dst[...], precision=_prec), attn_W1[...],
                precision=_prec) + attn_b1[...]
    v = jnp.dot(jnp.dot(h, W_src[...], precision=_prec), attn_W1[...],
                precision=_prec)
    p1 = jnp.dot(posb[...], pos_W1p[...], precision=_prec)
    vd_o[...] = jnp.concatenate([u, p1 + pos_b1[...]], axis=1)
    vsn_o[...] = -jnp.concatenate([v, p1], axis=1)
    msg_o[...] = jnp.dot(h, W_lin[...], precision=_prec)


def _prologue(x_pad, pos_pad, W_in, b_in, W_dst, W_src, attn_W1, attn_b1,
              pos_W1p, pos_b1, W_lin):
    nblk = NP_ // NB0
    row = pl.BlockSpec((NB0, None), lambda i: (i, 0))
    full = lambda a: pl.BlockSpec(a.shape, lambda i: tuple(0 for _ in a.shape))
    return pl.pallas_call(
        _prologue_body,
        grid=(nblk,),
        in_specs=[pl.BlockSpec((NB0, D), lambda i: (i, 0)),
                  pl.BlockSpec((NB0, 8), lambda i: (i, 0)),
                  full(W_in), full(b_in), full(W_dst), full(W_src),
                  full(attn_W1), full(attn_b1), full(pos_W1p), full(pos_b1),
                  full(W_lin)],
        out_specs=[pl.BlockSpec((NB0, D), lambda i: (i, 0))] * 3,
        out_shape=[jax.ShapeDtypeStruct((NP_, D), jnp.float32)] * 3,
    )(x_pad, pos_pad, W_in, b_in, W_dst, W_src, attn_W1, attn_b1,
      pos_W1p, pos_b1, W_lin)


def _gather_body(dstg, srcg, vd, vsn, msg, e1_o, msg_o,
                 dstb, srcb, e1b, msgb, semA, semB):
    c = lax.axis_index("c")
    s = lax.axis_index("s")
    wid = s * 2 + c
    rows_w = IDX_ROWS // 32
    base_row = wid * rows_w

    def body(i, _):
        r0 = base_row + i * (B1 // 128)
        e0 = r0 * 128
        pltpu.sync_copy(dstg.at[pl.ds(r0, B1 // 128)], dstb)
        pltpu.sync_copy(srcg.at[pl.ds(r0, B1 // 128)], srcb)
        for j in range(B1 // 128):
            pltpu.async_copy(vd.at[dstb.at[j]],
                             e1b.at[pl.ds(j * 128, 128)], semA)
            pltpu.async_copy(msg.at[srcb.at[j]],
                             msgb.at[pl.ds(j * 128, 128)], semB)
        for j in range(B1 // 128):
            pltpu.make_async_copy(vd.at[dstb.at[j]],
                                  e1b.at[pl.ds(j * 128, 128)], semA).wait()
            pltpu.make_async_copy(msg.at[srcb.at[j]],
                                  msgb.at[pl.ds(j * 128, 128)], semB).wait()
        for j in range(B1 // 128):
            pltpu.async_copy(vsn.at[srcb.at[j]],
                             e1b.at[pl.ds(j * 128, 128)], semA, add=True)
        pltpu.sync_copy(msgb, msg_o.at[pl.ds(e0, B1)])
        for j in range(B1 // 128):
            pltpu.make_async_copy(vsn.at[srcb.at[j]],
                                  e1b.at[pl.ds(j * 128, 128)], semA).wait()
        pltpu.sync_copy(e1b, e1_o.at[pl.ds(e0, B1)])
        return 0

    lax.fori_loop(0, NBLK, body, 0)


def _gather(dstg2, srcg2, vd, vsn, msg):
    mesh = plsc.VectorSubcoreMesh(core_axis_name="c", subcore_axis_name="s")
    f = pl.kernel(
        _gather_body,
        out_type=[jax.ShapeDtypeStruct((EP, D), jnp.float32),
                  jax.ShapeDtypeStruct((EP, D), jnp.float32)],
        mesh=mesh,
        scratch_types=[pltpu.VMEM((B1 // 128, 128), jnp.int32),
                       pltpu.VMEM((B1 // 128, 128), jnp.int32),
                       pltpu.VMEM((B1, D), jnp.float32),
                       pltpu.VMEM((B1, D), jnp.float32),
                       pltpu.SemaphoreType.DMA,
                       pltpu.SemaphoreType.DMA],
    )
    return f(dstg2, srcg2, vd, vsn, msg)


def _edge_body(e1_r, msg_r, W2a, b2a, W2p, b2p, pm_o):
    e1 = jax.nn.relu(e1_r[...])
    a = jax.nn.relu(jnp.dot(e1, W2a[...], precision=_prec) + b2a[...])
    dd = jax.nn.relu(jnp.dot(e1, W2p[...], precision=_prec) + b2p[...])
    p = jnp.exp(a)
    m = p * (msg_r[...] + dd)
    pm_o[...] = jnp.stack([p, m])


def _edge_mlp(e1_pre, msg_e, W2a, b2a, W2p, b2p):
    nblk = EP // NB0
    full = lambda a: pl.BlockSpec(a.shape, lambda i: tuple(0 for _ in a.shape))
    return pl.pallas_call(
        _edge_body,
        grid=(nblk,),
        in_specs=[pl.BlockSpec((NB0, D), lambda i: (i, 0)),
                  pl.BlockSpec((NB0, D), lambda i: (i, 0)),
                  full(W2a), full(b2a), full(W2p), full(b2p)],
        out_specs=[pl.BlockSpec((2, NB0, D), lambda i: (0, i, 0))],
        out_shape=[jax.ShapeDtypeStruct((2, EP, D), jnp.float32)],
    )(e1_pre, msg_e, W2a, b2a, W2p, b2p)[0]


def _scatter_body(dsts2, pm, zrows, acc_o, idxb, datab, acc, semA):
    c = lax.axis_index("c")
    s = lax.axis_index("s")
    rows_t = ACC_ROWS // 16
    pltpu.sync_copy(zrows.at[pl.ds(s * rows_t, rows_t)],
                    acc.at[pl.ds(s * rows_t, rows_t)])
    plsc.subcore_barrier()

    irows_t = IDX_ROWS // 16
    ib = B3 // 128

    def body(i, _):
        r0 = s * irows_t + i * ib
        e0 = r0 * 128
        pltpu.sync_copy(dsts2.at[pl.ds(r0, ib)], idxb)
        pltpu.sync_copy(pm.at[c, pl.ds(e0, B3)], datab)
        for j in range(ib):
            pltpu.async_copy(datab.at[pl.ds(j * 128, 128)],
                             acc.at[idxb.at[j]], semA, add=True)
        for j in range(ib):
            pltpu.make_async_copy(datab.at[pl.ds(j * 128, 128)],
                                  acc.at[idxb.at[j]], semA).wait()
        return 0

    lax.fori_loop(0, NBLK3, body, 0)
    plsc.subcore_barrier()
    pltpu.sync_copy(acc.at[pl.ds(s * rows_t, rows_t)],
                    acc_o.at[c, pl.ds(s * rows_t, rows_t)])


def _scatter(dsts2, pm, zrows):
    mesh = plsc.VectorSubcoreMesh(core_axis_name="c", subcore_axis_name="s")
    f = pl.kernel(
        _scatter_body,
        out_type=jax.ShapeDtypeStruct((2, ACC_ROWS, D), jnp.float32),
        mesh=mesh,
        scratch_types=[pltpu.VMEM((B3 // 128, 128), jnp.int32),
                       pltpu.VMEM((B3, D), jnp.float32),
                       pltpu.VMEM_SHARED((ACC_ROWS, D), jnp.float32),
                       pltpu.SemaphoreType.DMA],
    )
    return f(dsts2, pm, zrows)


def _epilogue_body(acc_r, W_out, b_out, out_o):
    den = acc_r[0]
    num = acc_r[1]
    o = num / (den + 1e-16)
    out_o[...] = jax.nn.relu(jnp.dot(o, W_out[...], precision=_prec)
                             + b_out[...])


def _epilogue(accs, W_out, b_out):
    blk = 1024
    nblk = ACC_ROWS // blk
    full = lambda a: pl.BlockSpec(a.shape, lambda i: tuple(0 for _ in a.shape))
    return pl.pallas_call(
        _epilogue_body,
        grid=(nblk,),
        in_specs=[pl.BlockSpec((2, blk, D), lambda i: (0, i, 0)),
                  full(W_out), full(b_out)],
        out_specs=[pl.BlockSpec((blk, D), lambda i: (i, 0))],
        out_shape=[jax.ShapeDtypeStruct((ACC_ROWS, D), jnp.float32)],
    )(accs, W_out, b_out)[0]


def kernel(x, pos, edge_index, W_in, b_in, W_out, b_out, W_lin, W_src, W_dst,
           pos_W1, pos_b1, pos_W2, pos_b2, attn_W1, attn_b1, attn_W2,
           attn_b2):
    idt = edge_index.dtype
    loops = jnp.arange(N, dtype=idt)
    padz = jnp.zeros((EP - EL,), idt)
    src = jnp.concatenate([edge_index[0], loops, padz])
    dstg = jnp.concatenate([edge_index[1], loops, padz])
    dsts = jnp.concatenate([edge_index[1], loops,
                            jnp.full((EP - EL,), N, idt)])
    srcg2 = src.reshape(IDX_ROWS, 128)
    dstg2 = dstg.reshape(IDX_ROWS, 128)
    dsts2 = dsts.reshape(IDX_ROWS, 128)

    x_pad = jnp.zeros((NP_, D), jnp.float32).at[:N].set(x)
    pos_pad = jnp.zeros((NP_, 8), jnp.float32).at[:N, :3].set(pos)
    pos_W1p = jnp.zeros((8, H), jnp.float32).at[:3].set(pos_W1)
    W2a = jnp.concatenate([attn_W2, jnp.zeros((H, D), jnp.float32)], axis=0)
    W2p = jnp.concatenate([jnp.zeros((H, D), jnp.float32), pos_W2], axis=0)

    vd, vsn, msg = _prologue(x_pad, pos_pad, W_in, b_in[None], W_dst, W_src,
                             attn_W1, attn_b1[None], pos_W1p, pos_b1[None],
                             W_lin)
    e1_pre, msg_e = _gather(dstg2, srcg2, vd, vsn, msg)
    pm = _edge_mlp(e1_pre, msg_e, W2a, attn_b2[None], W2p, pos_b2[None])
    accs = _scatter(dsts2, pm, jnp.zeros((ACC_ROWS, D), jnp.float32))
    out = _epilogue(accs, W_out, b_out[None])
    return out[:N]
